# baseline (speedup 1.0000x reference)
import functools

import jax
import jax.numpy as jnp
from jax import lax
from jax.experimental import pallas as pl
from jax.experimental.pallas import tpu as pltpu
from jax.experimental.pallas import tpu_sc as plsc

NC = 2
NS = 16
L = 16

EB = 64
K = 4


def _spmm_sc(xflat, pkd, attrb, n_nodes, n_edges, d_half):
  assert n_edges % EB == 0
  nblk = n_edges // EB
  blk_base, blk_extra = divmod(nblk, NS)
  rpt = (n_nodes // NS) & ~7
  tail = n_nodes - NS * rpt
  assert tail % 8 == 0
  zfull, zrem = divmod(rpt, EB)
  assert zrem % 8 == 0
  ngrp = d_half // L

  mesh = plsc.VectorSubcoreMesh(core_axis_name="c", subcore_axis_name="s")

  @functools.partial(
      pl.kernel,
      out_type=jax.ShapeDtypeStruct((NC, n_nodes, d_half), jnp.float32),
      mesh=mesh,
      scratch_types=[
          pltpu.VMEM((2 * K, 2, EB), jnp.int32),
          pltpu.VMEM((2 * K, 1, EB), jnp.float32),
          pltpu.VMEM((2 * K, EB), jnp.int32),
          pltpu.VMEM((K, EB, d_half), jnp.float32),
          pltpu.VMEM_SHARED((n_nodes, d_half), jnp.float32),
      ] + [pltpu.SemaphoreType.DMA] * (2 * K),
  )
  def spmm(xflat_hbm, pkd_hbm, attr_hbm, out_hbm, pkd_v, attr_v, idx_v, msg_v, agg_sh, *sems):
    gsem = sems[:K]
    ssem = sems[K:]
    c = lax.axis_index("c")
    s = lax.axis_index("s")

    def _zero_row(i, _):
      for g in range(ngrp):
        msg_v[0, i, pl.ds(g * L, L)] = jnp.zeros((L,), jnp.float32)
      return 0
    lax.fori_loop(0, EB, _zero_row, 0)

    r0 = s * rpt

    def _zero_agg(j, _):
      pltpu.sync_copy(msg_v.at[0], agg_sh.at[pl.ds(r0 + j * EB, EB)])
      return 0
    lax.fori_loop(0, zfull, _zero_agg, 0)
    if zrem:
      pltpu.sync_copy(msg_v.at[0, pl.ds(0, zrem)],
                      agg_sh.at[pl.ds(r0 + zfull * EB, zrem)])

    @pl.when(s == NS - 1)
    def _zero_tail():
      pltpu.sync_copy(msg_v.at[0, pl.ds(0, tail)],
                      agg_sh.at[pl.ds(NS * rpt, tail)])

    plsc.subcore_barrier()

    assert (blk_base - K) % (2 * K) == 0 and blk_base >= 3 * K
    b0 = s * blk_base
    niter = (blk_base - K) // (2 * K)

    def _compute_idx(j):
      def _grp(gi, _):
        idx_v[j, pl.ds(gi * L, L)] = pkd_v[j, 0, pl.ds(gi * L, L)] * 2 + c
        return 0
      lax.fori_loop(0, EB // L, _grp, 0)

    def _scale(m, j):
      def _body(gi, _):
        grp = attr_v[j, 0, pl.ds(gi * L, L)]
        for el in range(L):
          e = gi * L + el
          a = grp[el]
          for g in range(ngrp):
            msg_v[m, e, pl.ds(g * L, L)] = msg_v[m, e, pl.ds(g * L, L)] * a
        return 0
      lax.fori_loop(0, EB // L, _body, 0)

    def _fire_gather(m, j):
      pltpu.async_copy(xflat_hbm.at[idx_v.at[j]], msg_v.at[m], gsem[m])

    def _wait_gather(m, j):
      pltpu.make_async_copy(xflat_hbm.at[idx_v.at[j]], msg_v.at[m],
                            gsem[m]).wait()

    def _fire_scatter(m, j):
      pltpu.async_copy(msg_v.at[m], agg_sh.at[pkd_v.at[j, 1]], ssem[m],
                       add=True)

    def _wait_scatter(m, j):
      pltpu.make_async_copy(msg_v.at[m], agg_sh.at[pkd_v.at[j, 1]],
                            ssem[m]).wait()

    def _load_set(j0, b):
      pltpu.sync_copy(pkd_hbm.at[pl.ds(b, K)], pkd_v.at[pl.ds(j0, K)])
      pltpu.sync_copy(attr_hbm.at[pl.ds(b, K)], attr_v.at[pl.ds(j0, K)])
      for t in range(K):
        _compute_idx(j0 + t)

    _load_set(0, b0)
    for m in range(K):
      _fire_gather(m, m)

    def _roll(i, _):
      base = b0 + K + i * 2 * K
      _load_set(K, base)
      for m in range(K):
        _wait_gather(m, m); _scale(m, m); _fire_scatter(m, m)
      for m in range(K):
        _wait_scatter(m, m); _fire_gather(m, K + m)
      _load_set(0, base + K)
      for m in range(K):
        _wait_gather(m, K + m); _scale(m, K + m); _fire_scatter(m, K + m)
      for m in range(K):
        _wait_scatter(m, K + m); _fire_gather(m, m)
      return 0

    lax.fori_loop(0, niter, _roll, 0)

    for m in range(K):
      _wait_gather(m, m); _scale(m, m); _fire_scatter(m, m)
    for m in range(K):
      _wait_scatter(m, m)

    @pl.when(s < blk_extra)
    def _leftover():
      b = NS * blk_base + s
      pltpu.sync_copy(pkd_hbm.at[pl.ds(b, 1)], pkd_v.at[pl.ds(0, 1)])
      pltpu.sync_copy(attr_hbm.at[pl.ds(b, 1)], attr_v.at[pl.ds(0, 1)])
      _compute_idx(0)
      pltpu.sync_copy(xflat_hbm.at[idx_v.at[0]], msg_v.at[0])
      _scale(0, 0)
      pltpu.sync_copy(msg_v.at[0], agg_sh.at[pkd_v.at[0, 1]], add=True)

    plsc.subcore_barrier()

    pltpu.sync_copy(agg_sh.at[pl.ds(r0, rpt)],
                    out_hbm.at[c, pl.ds(r0, rpt)])

    @pl.when(s == NS - 1)
    def _write_tail():
      pltpu.sync_copy(agg_sh.at[pl.ds(NS * rpt, tail)],
                      out_hbm.at[c, pl.ds(NS * rpt, tail)])

  return spmm(xflat, pkd, attrb)


def _dense_tc(agg2, w2, bias2d, n_nodes, d_out, block_m):
  d_half = agg2.shape[2]

  def body(a_ref, w_ref, b_ref, o_ref):
    y = jax.lax.dot_general(
        a_ref[0], w_ref[0], (((1,), (0,)), ((), ())),
        precision=lax.Precision.DEFAULT,
        preferred_element_type=jnp.float32)
    y = y + jax.lax.dot_general(
        a_ref[1], w_ref[1], (((1,), (0,)), ((), ())),
        precision=lax.Precision.DEFAULT,
        preferred_element_type=jnp.float32)
    y = y + b_ref[...]
    n2 = jnp.sum(y * y, axis=-1, keepdims=True)
    denom = jnp.maximum(jnp.sqrt(n2), 1e-12)
    o_ref[...] = y / denom

  grid = (n_nodes // block_m,)
  return pl.pallas_call(
      body,
      grid=grid,
      in_specs=[
          pl.BlockSpec((NC, block_m, d_half), lambda i: (0, i, 0)),
          pl.BlockSpec((NC, d_half, d_out), lambda i: (0, 0, 0)),
          pl.BlockSpec((1, d_out), lambda i: (0, 0)),
      ],
      out_specs=pl.BlockSpec((block_m, d_out), lambda i: (i, 0)),
      out_shape=jax.ShapeDtypeStruct((n_nodes, d_out), jnp.float32),
  )(agg2, w2, bias2d)


@jax.jit
def kernel(x, edge_index, edge_attr, W, b):
  n_nodes, d_in = x.shape
  n_edges = edge_index.shape[1]
  d_out = W.shape[1]
  d_half = d_in // NC

  xflat = x.reshape(n_nodes * NC, d_half)
  row = edge_index[0].astype(jnp.int32)
  col = edge_index[1].astype(jnp.int32)
  pkd = jnp.stack([col.reshape(-1, EB), row.reshape(-1, EB)], axis=1)
  attrb = edge_attr.astype(jnp.float32).reshape(-1, 1, EB)

  agg2 = _spmm_sc(xflat, pkd, attrb, n_nodes, n_edges, d_half)

  w2 = W.reshape(NC, d_half, d_out)
  bias2d = b.reshape(1, d_out)
  return _dense_tc(agg2, w2, bias2d, n_nodes, d_out, block_m=2000)


# speedup vs baseline: 1.0889x; 1.0889x over previous
import functools

import jax
import jax.numpy as jnp
from jax import lax
from jax.experimental import pallas as pl
from jax.experimental.pallas import tpu as pltpu
from jax.experimental.pallas import tpu_sc as plsc

NC = 2
NS = 16
L = 16

EB = 80
K = 3


def _spmm_sc(xflat, pkd, attrb, n_nodes, n_edges, d_half):
  assert n_edges % EB == 0
  nblk = n_edges // EB
  blk_raw = nblk // NS
  blk_eff = ((blk_raw - K) // (2 * K)) * 2 * K + K
  nleft = nblk - NS * blk_eff
  nl_base, nl_extra = divmod(nleft, NS)
  rpt = (n_nodes // NS) & ~7
  tail = n_nodes - NS * rpt
  assert tail % 8 == 0
  zfull, zrem = divmod(rpt, EB)
  assert zrem % 8 == 0
  ngrp = d_half // L

  mesh = plsc.VectorSubcoreMesh(core_axis_name="c", subcore_axis_name="s")

  @functools.partial(
      pl.kernel,
      out_type=jax.ShapeDtypeStruct((NC, n_nodes, d_half), jnp.float32),
      mesh=mesh,
      scratch_types=[
          pltpu.VMEM((2 * K, 2, EB), jnp.int32),
          pltpu.VMEM((2 * K, 1, EB), jnp.float32),
          pltpu.VMEM((2 * K, EB), jnp.int32),
          pltpu.VMEM((K, EB, d_half), jnp.float32),
          pltpu.VMEM_SHARED((n_nodes, d_half), jnp.float32),
      ] + [pltpu.SemaphoreType.DMA] * (2 * K),
  )
  def spmm(xflat_hbm, pkd_hbm, attr_hbm, out_hbm, pkd_v, attr_v, idx_v, msg_v, agg_sh, *sems):
    gsem = sems[:K]
    ssem = sems[K:]
    c = lax.axis_index("c")
    s = lax.axis_index("s")

    def _zero_row(i, _):
      for g in range(ngrp):
        msg_v[0, i, pl.ds(g * L, L)] = jnp.zeros((L,), jnp.float32)
      return 0
    lax.fori_loop(0, EB, _zero_row, 0)

    r0 = s * rpt

    def _zero_agg(j, _):
      pltpu.sync_copy(msg_v.at[0], agg_sh.at[pl.ds(r0 + j * EB, EB)])
      return 0
    lax.fori_loop(0, zfull, _zero_agg, 0)
    if zrem:
      pltpu.sync_copy(msg_v.at[0, pl.ds(0, zrem)],
                      agg_sh.at[pl.ds(r0 + zfull * EB, zrem)])

    @pl.when(s == NS - 1)
    def _zero_tail():
      pltpu.sync_copy(msg_v.at[0, pl.ds(0, tail)],
                      agg_sh.at[pl.ds(NS * rpt, tail)])

    plsc.subcore_barrier()

    b0 = s * blk_eff
    niter = (blk_eff - K) // (2 * K)

    def _compute_idx(j):
      def _grp(gi, _):
        idx_v[j, pl.ds(gi * L, L)] = pkd_v[j, 0, pl.ds(gi * L, L)] * 2 + c
        return 0
      lax.fori_loop(0, EB // L, _grp, 0)

    def _scale(m, j):
      def _body(gi, _):
        grp = attr_v[j, 0, pl.ds(gi * L, L)]
        for el in range(L):
          e = gi * L + el
          a = grp[el]
          for g in range(ngrp):
            msg_v[m, e, pl.ds(g * L, L)] = msg_v[m, e, pl.ds(g * L, L)] * a
        return 0
      lax.fori_loop(0, EB // L, _body, 0)

    def _fire_gather(m, j):
      pltpu.async_copy(xflat_hbm.at[idx_v.at[j]], msg_v.at[m], gsem[m])

    def _wait_gather(m, j):
      pltpu.make_async_copy(xflat_hbm.at[idx_v.at[j]], msg_v.at[m],
                            gsem[m]).wait()

    def _fire_scatter(m, j):
      pltpu.async_copy(msg_v.at[m], agg_sh.at[pkd_v.at[j, 1]], ssem[m],
                       add=True)

    def _wait_scatter(m, j):
      pltpu.make_async_copy(msg_v.at[m], agg_sh.at[pkd_v.at[j, 1]],
                            ssem[m]).wait()

    def _load_set(j0, b):
      pltpu.sync_copy(pkd_hbm.at[pl.ds(b, K)], pkd_v.at[pl.ds(j0, K)])
      pltpu.sync_copy(attr_hbm.at[pl.ds(b, K)], attr_v.at[pl.ds(j0, K)])
      for t in range(K):
        _compute_idx(j0 + t)

    _load_set(0, b0)
    for m in range(K):
      _fire_gather(m, m)

    def _roll(i, _):
      base = b0 + K + i * 2 * K
      _load_set(K, base)
      for m in range(K):
        _wait_gather(m, m); _scale(m, m); _fire_scatter(m, m)
      for m in range(K):
        _wait_scatter(m, m); _fire_gather(m, K + m)
      _load_set(0, base + K)
      for m in range(K):
        _wait_gather(m, K + m); _scale(m, K + m); _fire_scatter(m, K + m)
      for m in range(K):
        _wait_scatter(m, K + m); _fire_gather(m, m)
      return 0

    lax.fori_loop(0, niter, _roll, 0)

    for m in range(K):
      _wait_gather(m, m); _scale(m, m); _fire_scatter(m, m)
    for m in range(K):
      _wait_scatter(m, m)

    lb0 = NS * blk_eff + s * nl_base + jnp.minimum(s, nl_extra)
    nl = nl_base + jnp.where(s < nl_extra, 1, 0)

    def _leftover(t, _):
      b = lb0 + t
      pltpu.sync_copy(pkd_hbm.at[pl.ds(b, 1)], pkd_v.at[pl.ds(0, 1)])
      pltpu.sync_copy(attr_hbm.at[pl.ds(b, 1)], attr_v.at[pl.ds(0, 1)])
      _compute_idx(0)
      pltpu.sync_copy(xflat_hbm.at[idx_v.at[0]], msg_v.at[0])
      _scale(0, 0)
      pltpu.sync_copy(msg_v.at[0], agg_sh.at[pkd_v.at[0, 1]], add=True)
      return 0

    lax.fori_loop(0, nl, _leftover, 0)

    plsc.subcore_barrier()

    pltpu.sync_copy(agg_sh.at[pl.ds(r0, rpt)],
                    out_hbm.at[c, pl.ds(r0, rpt)])

    @pl.when(s == NS - 1)
    def _write_tail():
      pltpu.sync_copy(agg_sh.at[pl.ds(NS * rpt, tail)],
                      out_hbm.at[c, pl.ds(NS * rpt, tail)])

  return spmm(xflat, pkd, attrb)


def _dense_tc(agg2, w2, bias2d, n_nodes, d_out, block_m):
  d_half = agg2.shape[2]

  def body(a_ref, w_ref, b_ref, o_ref):
    y = jax.lax.dot_general(
        a_ref[0], w_ref[0], (((1,), (0,)), ((), ())),
        precision=lax.Precision.DEFAULT,
        preferred_element_type=jnp.float32)
    y = y + jax.lax.dot_general(
        a_ref[1], w_ref[1], (((1,), (0,)), ((), ())),
        precision=lax.Precision.DEFAULT,
        preferred_element_type=jnp.float32)
    y = y + b_ref[...]
    n2 = jnp.sum(y * y, axis=-1, keepdims=True)
    denom = jnp.maximum(jnp.sqrt(n2), 1e-12)
    o_ref[...] = y / denom

  grid = (n_nodes // block_m,)
  return pl.pallas_call(
      body,
      grid=grid,
      in_specs=[
          pl.BlockSpec((NC, block_m, d_half), lambda i: (0, i, 0)),
          pl.BlockSpec((NC, d_half, d_out), lambda i: (0, 0, 0)),
          pl.BlockSpec((1, d_out), lambda i: (0, 0)),
      ],
      out_specs=pl.BlockSpec((block_m, d_out), lambda i: (i, 0)),
      out_shape=jax.ShapeDtypeStruct((n_nodes, d_out), jnp.float32),
  )(agg2, w2, bias2d)


@jax.jit
def kernel(x, edge_index, edge_attr, W, b):
  n_nodes, d_in = x.shape
  n_edges = edge_index.shape[1]
  d_out = W.shape[1]
  d_half = d_in // NC

  xflat = x.reshape(n_nodes * NC, d_half)
  row = edge_index[0].astype(jnp.int32)
  col = edge_index[1].astype(jnp.int32)
  pkd = jnp.stack([col.reshape(-1, EB), row.reshape(-1, EB)], axis=1)
  attrb = edge_attr.astype(jnp.float32).reshape(-1, 1, EB)

  agg2 = _spmm_sc(xflat, pkd, attrb, n_nodes, n_edges, d_half)

  w2 = W.reshape(NC, d_half, d_out)
  bias2d = b.reshape(1, d_out)
  return _dense_tc(agg2, w2, bias2d, n_nodes, d_out, block_m=2000)


# speedup vs baseline: 1.2303x; 1.1298x over previous
import functools

import jax
import jax.numpy as jnp
from jax import lax
from jax.experimental import pallas as pl
from jax.experimental.pallas import tpu as pltpu
from jax.experimental.pallas import tpu_sc as plsc

NC = 2
NS = 16
L = 16

EB = 128
K = 2


def _spmm_sc(xflat, pkd, attrb, n_nodes, n_edges, d_half):
  assert n_edges % EB == 0
  nblk = n_edges // EB
  blk_base, blk_extra = divmod(nblk, NS)
  rpt = (n_nodes // NS) & ~7
  tail = n_nodes - NS * rpt
  assert tail % 8 == 0
  zfull, zrem = divmod(rpt, EB)
  assert zrem % 8 == 0
  ngrp = d_half // L

  mesh = plsc.VectorSubcoreMesh(core_axis_name="c", subcore_axis_name="s")

  @functools.partial(
      pl.kernel,
      out_type=jax.ShapeDtypeStruct((NC, n_nodes, d_half), jnp.float32),
      mesh=mesh,
      scratch_types=[
          pltpu.VMEM((2 * K, 2, EB), jnp.int32),
          pltpu.VMEM((2 * K, 1, EB), jnp.float32),
          pltpu.VMEM((2 * K, EB), jnp.int32),
          pltpu.VMEM((K, EB, d_half), jnp.float32),
          pltpu.VMEM_SHARED((n_nodes, d_half), jnp.float32),
      ] + [pltpu.SemaphoreType.DMA] * (2 * K + 2),
  )
  def spmm(xflat_hbm, pkd_hbm, attr_hbm, out_hbm, pkd_v, attr_v, idx_v, msg_v, agg_sh, *sems):
    gsem = sems[:K]
    ssem = sems[K:2 * K]
    lsem = sems[2 * K:]
    c = lax.axis_index("c")
    s = lax.axis_index("s")

    def _zero_row(i, _):
      for g in range(ngrp):
        msg_v[0, i, pl.ds(g * L, L)] = jnp.zeros((L,), jnp.float32)
      return 0
    lax.fori_loop(0, EB, _zero_row, 0)

    r0 = s * rpt

    def _zero_agg(j, _):
      pltpu.sync_copy(msg_v.at[0], agg_sh.at[pl.ds(r0 + j * EB, EB)])
      return 0
    lax.fori_loop(0, zfull, _zero_agg, 0)
    if zrem:
      pltpu.sync_copy(msg_v.at[0, pl.ds(0, zrem)],
                      agg_sh.at[pl.ds(r0 + zfull * EB, zrem)])

    @pl.when(s == NS - 1)
    def _zero_tail():
      pltpu.sync_copy(msg_v.at[0, pl.ds(0, tail)],
                      agg_sh.at[pl.ds(NS * rpt, tail)])

    plsc.subcore_barrier()

    assert blk_base % 4 == 2 and blk_base >= 6
    b0 = s * blk_base
    niter = (blk_base - 2) // 4

    def _compute_idx(j):
      def _grp(gi, _):
        idx_v[j, pl.ds(gi * L, L)] = pkd_v[j, 0, pl.ds(gi * L, L)] * 2 + c
        return 0
      lax.fori_loop(0, EB // L, _grp, 0)

    def _scale(m, j):
      def _body(gi, _):
        grp = attr_v[j, 0, pl.ds(gi * L, L)]
        for el in range(L):
          e = gi * L + el
          a = grp[el]
          for g in range(ngrp):
            msg_v[m, e, pl.ds(g * L, L)] = msg_v[m, e, pl.ds(g * L, L)] * a
        return 0
      lax.fori_loop(0, EB // L, _body, 0)

    def _fire_gather(m, j):
      pltpu.async_copy(xflat_hbm.at[idx_v.at[j]], msg_v.at[m], gsem[m])

    def _wait_gather(m, j):
      pltpu.make_async_copy(xflat_hbm.at[idx_v.at[j]], msg_v.at[m],
                            gsem[m]).wait()

    def _fire_scatter(m, j):
      pltpu.async_copy(msg_v.at[m], agg_sh.at[pkd_v.at[j, 1]], ssem[m],
                       add=True)

    def _wait_scatter(m, j):
      pltpu.make_async_copy(msg_v.at[m], agg_sh.at[pkd_v.at[j, 1]],
                            ssem[m]).wait()

    def _fire_load(j2, b):
      pltpu.async_copy(pkd_hbm.at[pl.ds(b, 2)], pkd_v.at[pl.ds(j2, 2)],
                       lsem[0])
      pltpu.async_copy(attr_hbm.at[pl.ds(b, 2)], attr_v.at[pl.ds(j2, 2)],
                       lsem[1])

    def _wait_load(j2, b):
      pltpu.make_async_copy(pkd_hbm.at[pl.ds(b, 2)], pkd_v.at[pl.ds(j2, 2)],
                            lsem[0]).wait()
      pltpu.make_async_copy(attr_hbm.at[pl.ds(b, 2)], attr_v.at[pl.ds(j2, 2)],
                            lsem[1]).wait()

    _fire_load(0, b0)
    _wait_load(0, b0)
    _compute_idx(0)
    _compute_idx(1)
    _fire_gather(0, 0)
    _fire_gather(1, 1)

    def _roll(i, _):
      base = b0 + 2 + i * 4
      _fire_load(2, base)
      _wait_gather(0, 0); _scale(0, 0); _fire_scatter(0, 0)
      _wait_gather(1, 1); _scale(1, 1); _fire_scatter(1, 1)
      _wait_load(2, base)
      _compute_idx(2); _compute_idx(3)
      _wait_scatter(0, 0); _fire_gather(0, 2)
      _wait_scatter(1, 1); _fire_gather(1, 3)
      _fire_load(0, base + 2)
      _wait_gather(0, 2); _scale(0, 2); _fire_scatter(0, 2)
      _wait_gather(1, 3); _scale(1, 3); _fire_scatter(1, 3)
      _wait_load(0, base + 2)
      _compute_idx(0); _compute_idx(1)
      _wait_scatter(0, 2); _fire_gather(0, 0)
      _wait_scatter(1, 3); _fire_gather(1, 1)
      return 0

    lax.fori_loop(0, niter, _roll, 0)

    _wait_gather(0, 0); _scale(0, 0); _fire_scatter(0, 0)
    _wait_gather(1, 1); _scale(1, 1); _fire_scatter(1, 1)
    _wait_scatter(0, 0)
    _wait_scatter(1, 1)

    @pl.when(s < blk_extra)
    def _leftover():
      b = NS * blk_base + s
      pltpu.sync_copy(pkd_hbm.at[pl.ds(b, 1)], pkd_v.at[pl.ds(0, 1)])
      pltpu.sync_copy(attr_hbm.at[pl.ds(b, 1)], attr_v.at[pl.ds(0, 1)])
      _compute_idx(0)
      pltpu.sync_copy(xflat_hbm.at[idx_v.at[0]], msg_v.at[0])
      _scale(0, 0)
      pltpu.sync_copy(msg_v.at[0], agg_sh.at[pkd_v.at[0, 1]], add=True)

    plsc.subcore_barrier()

    pltpu.sync_copy(agg_sh.at[pl.ds(r0, rpt)],
                    out_hbm.at[c, pl.ds(r0, rpt)])

    @pl.when(s == NS - 1)
    def _write_tail():
      pltpu.sync_copy(agg_sh.at[pl.ds(NS * rpt, tail)],
                      out_hbm.at[c, pl.ds(NS * rpt, tail)])

  return spmm(xflat, pkd, attrb)


def _dense_tc(agg2, w2, bias2d, n_nodes, d_out, block_m):
  d_half = agg2.shape[2]

  def body(a_ref, w_ref, b_ref, o_ref):
    y = jax.lax.dot_general(
        a_ref[0], w_ref[0], (((1,), (0,)), ((), ())),
        precision=lax.Precision.DEFAULT,
        preferred_element_type=jnp.float32)
    y = y + jax.lax.dot_general(
        a_ref[1], w_ref[1], (((1,), (0,)), ((), ())),
        precision=lax.Precision.DEFAULT,
        preferred_element_type=jnp.float32)
    y = y + b_ref[...]
    n2 = jnp.sum(y * y, axis=-1, keepdims=True)
    denom = jnp.maximum(jnp.sqrt(n2), 1e-12)
    o_ref[...] = y / denom

  grid = (n_nodes // block_m,)
  return pl.pallas_call(
      body,
      grid=grid,
      in_specs=[
          pl.BlockSpec((NC, block_m, d_half), lambda i: (0, i, 0)),
          pl.BlockSpec((NC, d_half, d_out), lambda i: (0, 0, 0)),
          pl.BlockSpec((1, d_out), lambda i: (0, 0)),
      ],
      out_specs=pl.BlockSpec((block_m, d_out), lambda i: (i, 0)),
      out_shape=jax.ShapeDtypeStruct((n_nodes, d_out), jnp.float32),
  )(agg2, w2, bias2d)


@jax.jit
def kernel(x, edge_index, edge_attr, W, b):
  n_nodes, d_in = x.shape
  n_edges = edge_index.shape[1]
  d_out = W.shape[1]
  d_half = d_in // NC

  xflat = x.reshape(n_nodes * NC, d_half)
  row = edge_index[0].astype(jnp.int32)
  col = edge_index[1].astype(jnp.int32)
  pkd = jnp.stack([col.reshape(-1, EB), row.reshape(-1, EB)], axis=1)
  attrb = edge_attr.astype(jnp.float32).reshape(-1, 1, EB)

  agg2 = _spmm_sc(xflat, pkd, attrb, n_nodes, n_edges, d_half)

  w2 = W.reshape(NC, d_half, d_out)
  bias2d = b.reshape(1, d_out)
  return _dense_tc(agg2, w2, bias2d, n_nodes, d_out, block_m=2000)


# speedup vs baseline: 1.2308x; 1.0004x over previous
import functools

import jax
import jax.numpy as jnp
from jax import lax
from jax.experimental import pallas as pl
from jax.experimental.pallas import tpu as pltpu
from jax.experimental.pallas import tpu_sc as plsc

NC = 2
NS = 16
L = 16

EB = 128
K = 2


def _spmm_sc(xflat, pkd, attrb, n_nodes, n_edges, d_half):
  assert n_edges % EB == 0
  nblk = n_edges // EB
  blk_base, blk_extra = divmod(nblk, NS)
  rpt = (n_nodes // NS) & ~7
  tail = n_nodes - NS * rpt
  assert tail % 8 == 0
  zfull, zrem = divmod(rpt, EB)
  assert zrem % 8 == 0
  ngrp = d_half // L

  mesh = plsc.VectorSubcoreMesh(core_axis_name="c", subcore_axis_name="s")

  @functools.partial(
      pl.kernel,
      out_type=jax.ShapeDtypeStruct((NC, n_nodes, d_half), jnp.float32),
      mesh=mesh,
      scratch_types=[
          pltpu.VMEM((2 * K, 2, EB), jnp.int32),
          pltpu.VMEM((2 * K, 1, EB), jnp.float32),
          pltpu.VMEM((2 * K, EB), jnp.int32),
          pltpu.VMEM((K, EB, d_half), jnp.float32),
          pltpu.VMEM_SHARED((n_nodes, d_half), jnp.float32),
      ] + [pltpu.SemaphoreType.DMA] * (2 * K + 2),
  )
  def spmm(xflat_hbm, pkd_hbm, attr_hbm, out_hbm, pkd_v, attr_v, idx_v, msg_v, agg_sh, *sems):
    gsem = sems[:K]
    ssem = sems[K:2 * K]
    lsem = sems[2 * K:]
    c = lax.axis_index("c")
    s = lax.axis_index("s")

    def _zero_row(i, _):
      for g in range(ngrp):
        msg_v[0, i, pl.ds(g * L, L)] = jnp.zeros((L,), jnp.float32)
      return 0
    lax.fori_loop(0, EB, _zero_row, 0)

    r0 = s * rpt

    def _zero_agg(j, _):
      pltpu.sync_copy(msg_v.at[0], agg_sh.at[pl.ds(r0 + j * EB, EB)])
      return 0
    lax.fori_loop(0, zfull, _zero_agg, 0)
    if zrem:
      pltpu.sync_copy(msg_v.at[0, pl.ds(0, zrem)],
                      agg_sh.at[pl.ds(r0 + zfull * EB, zrem)])

    @pl.when(s == NS - 1)
    def _zero_tail():
      pltpu.sync_copy(msg_v.at[0, pl.ds(0, tail)],
                      agg_sh.at[pl.ds(NS * rpt, tail)])

    plsc.subcore_barrier()

    assert blk_base % 4 == 2 and blk_base >= 6
    b0 = s * blk_base
    niter = (blk_base - 2) // 4

    def _compute_idx(j):
      def _grp(gi, _):
        idx_v[j, pl.ds(gi * L, L)] = pkd_v[j, 0, pl.ds(gi * L, L)] * 2 + c
        return 0
      lax.fori_loop(0, EB // L, _grp, 0)

    def _scale(m, j):
      def _body(gi, _):
        grp = attr_v[j, 0, pl.ds(gi * L, L)]
        for el in range(L):
          e = gi * L + el
          a = grp[el]
          for g in range(ngrp):
            msg_v[m, e, pl.ds(g * L, L)] = msg_v[m, e, pl.ds(g * L, L)] * a
        return 0
      lax.fori_loop(0, EB // L, _body, 0)

    def _fire_gather(m, j):
      pltpu.async_copy(xflat_hbm.at[idx_v.at[j]], msg_v.at[m], gsem[m])

    def _wait_gather(m, j):
      pltpu.make_async_copy(xflat_hbm.at[idx_v.at[j]], msg_v.at[m],
                            gsem[m]).wait()

    def _fire_scatter(m, j):
      pltpu.async_copy(msg_v.at[m], agg_sh.at[pkd_v.at[j, 1]], ssem[m],
                       add=True)

    def _wait_scatter(m, j):
      pltpu.make_async_copy(msg_v.at[m], agg_sh.at[pkd_v.at[j, 1]],
                            ssem[m]).wait()

    def _fire_load(j2, b):
      pltpu.async_copy(pkd_hbm.at[pl.ds(b, 2)], pkd_v.at[pl.ds(j2, 2)],
                       lsem[0])
      pltpu.async_copy(attr_hbm.at[pl.ds(b, 2)], attr_v.at[pl.ds(j2, 2)],
                       lsem[1])

    def _wait_load(j2, b):
      pltpu.make_async_copy(pkd_hbm.at[pl.ds(b, 2)], pkd_v.at[pl.ds(j2, 2)],
                            lsem[0]).wait()
      pltpu.make_async_copy(attr_hbm.at[pl.ds(b, 2)], attr_v.at[pl.ds(j2, 2)],
                            lsem[1]).wait()

    _fire_load(0, b0)
    _wait_load(0, b0)
    _compute_idx(0)
    _compute_idx(1)
    _fire_gather(0, 0)
    _fire_gather(1, 1)

    def _roll(i, _):
      base = b0 + 2 + i * 4
      _fire_load(2, base)
      _wait_gather(0, 0); _scale(0, 0); _fire_scatter(0, 0)
      _wait_gather(1, 1); _scale(1, 1); _fire_scatter(1, 1)
      _wait_load(2, base)
      _compute_idx(2); _compute_idx(3)
      _wait_scatter(0, 0); _fire_gather(0, 2)
      _wait_scatter(1, 1); _fire_gather(1, 3)
      _fire_load(0, base + 2)
      _wait_gather(0, 2); _scale(0, 2); _fire_scatter(0, 2)
      _wait_gather(1, 3); _scale(1, 3); _fire_scatter(1, 3)
      _wait_load(0, base + 2)
      _compute_idx(0); _compute_idx(1)
      _wait_scatter(0, 2); _fire_gather(0, 0)
      _wait_scatter(1, 3); _fire_gather(1, 1)
      return 0

    lax.fori_loop(0, niter, _roll, 0)

    _wait_gather(0, 0); _scale(0, 0); _fire_scatter(0, 0)
    _wait_gather(1, 1); _scale(1, 1); _fire_scatter(1, 1)
    _wait_scatter(0, 0)
    _wait_scatter(1, 1)

    @pl.when(s < blk_extra)
    def _leftover():
      b = NS * blk_base + s
      pltpu.sync_copy(pkd_hbm.at[pl.ds(b, 1)], pkd_v.at[pl.ds(0, 1)])
      pltpu.sync_copy(attr_hbm.at[pl.ds(b, 1)], attr_v.at[pl.ds(0, 1)])
      _compute_idx(0)
      pltpu.sync_copy(xflat_hbm.at[idx_v.at[0]], msg_v.at[0])
      _scale(0, 0)
      pltpu.sync_copy(msg_v.at[0], agg_sh.at[pkd_v.at[0, 1]], add=True)

    plsc.subcore_barrier()

    pltpu.sync_copy(agg_sh.at[pl.ds(r0, rpt)],
                    out_hbm.at[c, pl.ds(r0, rpt)])

    @pl.when(s == NS - 1)
    def _write_tail():
      pltpu.sync_copy(agg_sh.at[pl.ds(NS * rpt, tail)],
                      out_hbm.at[c, pl.ds(NS * rpt, tail)])

  return spmm(xflat, pkd, attrb)


def _dense_tc(agg2, w2, bias2d, n_nodes, d_out, block_m):
  d_half = agg2.shape[2]

  def body(a_ref, w_ref, b_ref, o_ref):
    y = jax.lax.dot_general(
        a_ref[0].astype(jnp.bfloat16), w_ref[0].astype(jnp.bfloat16),
        (((1,), (0,)), ((), ())),
        preferred_element_type=jnp.float32)
    y = y + jax.lax.dot_general(
        a_ref[1].astype(jnp.bfloat16), w_ref[1].astype(jnp.bfloat16),
        (((1,), (0,)), ((), ())),
        preferred_element_type=jnp.float32)
    y = y + b_ref[...]
    n2 = jnp.sum(y * y, axis=-1, keepdims=True)
    denom = jnp.maximum(jnp.sqrt(n2), 1e-12)
    o_ref[...] = y / denom

  grid = (n_nodes // block_m,)
  return pl.pallas_call(
      body,
      grid=grid,
      in_specs=[
          pl.BlockSpec((NC, block_m, d_half), lambda i: (0, i, 0)),
          pl.BlockSpec((NC, d_half, d_out), lambda i: (0, 0, 0)),
          pl.BlockSpec((1, d_out), lambda i: (0, 0)),
      ],
      out_specs=pl.BlockSpec((block_m, d_out), lambda i: (i, 0)),
      out_shape=jax.ShapeDtypeStruct((n_nodes, d_out), jnp.float32),
  )(agg2, w2, bias2d)


@jax.jit
def kernel(x, edge_index, edge_attr, W, b):
  n_nodes, d_in = x.shape
  n_edges = edge_index.shape[1]
  d_out = W.shape[1]
  d_half = d_in // NC

  xflat = x.reshape(n_nodes * NC, d_half)
  row = edge_index[0].astype(jnp.int32)
  col = edge_index[1].astype(jnp.int32)
  pkd = jnp.stack([col.reshape(-1, EB), row.reshape(-1, EB)], axis=1)
  attrb = edge_attr.astype(jnp.float32).reshape(-1, 1, EB)

  agg2 = _spmm_sc(xflat, pkd, attrb, n_nodes, n_edges, d_half)

  w2 = W.reshape(NC, d_half, d_out)
  bias2d = b.reshape(1, d_out)
  return _dense_tc(agg2, w2, bias2d, n_nodes, d_out, block_m=2000)


# speedup vs baseline: 1.2336x; 1.0023x over previous
import functools

import jax
import jax.numpy as jnp
from jax import lax
from jax.experimental import pallas as pl
from jax.experimental.pallas import tpu as pltpu
from jax.experimental.pallas import tpu_sc as plsc

NC = 2
NS = 16
L = 16

EB = 128
K = 2


def _spmm_sc(xflat, pkd, attrb, n_nodes, n_edges, d_half):
  assert n_edges % EB == 0
  nblk = n_edges // EB
  blk_base, blk_extra = divmod(nblk, NS)
  rpt = (n_nodes // NS) & ~7
  tail = n_nodes - NS * rpt
  assert tail % 8 == 0
  zfull, zrem = divmod(rpt, EB)
  assert zrem % 8 == 0
  ngrp = d_half // L

  mesh = plsc.VectorSubcoreMesh(core_axis_name="c", subcore_axis_name="s")

  @functools.partial(
      pl.kernel,
      out_type=jax.ShapeDtypeStruct((NC, n_nodes, d_half), jnp.float32),
      mesh=mesh,
      scratch_types=[
          pltpu.VMEM((2 * K, 2, EB), jnp.int32),
          pltpu.VMEM((2 * K, 1, EB), jnp.float32),
          pltpu.VMEM((2 * K, EB), jnp.int32),
          pltpu.VMEM((K, EB, d_half), jnp.float32),
          pltpu.VMEM_SHARED((n_nodes, d_half), jnp.float32),
      ] + [pltpu.SemaphoreType.DMA] * (2 * K + 2),
  )
  def spmm(xflat_hbm, pkd_hbm, attr_hbm, out_hbm, pkd_v, attr_v, idx_v, msg_v, agg_sh, *sems):
    gsem = sems[:K]
    ssem = sems[K:2 * K]
    lsem = sems[2 * K:]
    c = lax.axis_index("c")
    s = lax.axis_index("s")

    def _zero_row(i, _):
      for g in range(ngrp):
        msg_v[0, i, pl.ds(g * L, L)] = jnp.zeros((L,), jnp.float32)
      return 0
    lax.fori_loop(0, EB, _zero_row, 0)

    r0 = s * rpt

    def _zero_agg(j, _):
      pltpu.sync_copy(msg_v.at[0], agg_sh.at[pl.ds(r0 + j * EB, EB)])
      return 0
    lax.fori_loop(0, zfull, _zero_agg, 0)
    if zrem:
      pltpu.sync_copy(msg_v.at[0, pl.ds(0, zrem)],
                      agg_sh.at[pl.ds(r0 + zfull * EB, zrem)])

    @pl.when(s == NS - 1)
    def _zero_tail():
      pltpu.sync_copy(msg_v.at[0, pl.ds(0, tail)],
                      agg_sh.at[pl.ds(NS * rpt, tail)])

    plsc.subcore_barrier()

    assert blk_base % 4 == 2 and blk_base >= 6
    b0 = s * blk_base
    niter = (blk_base - 2) // 4

    def _compute_idx(j):
      def _grp(gi, _):
        idx_v[j, pl.ds(gi * L, L)] = pkd_v[j, 0, pl.ds(gi * L, L)] * 2 + c
        return 0
      lax.fori_loop(0, EB // L, _grp, 0)

    def _scale(m, j):
      def _body(gi, _):
        grp = attr_v[j, 0, pl.ds(gi * L, L)]
        for el in range(L):
          e = gi * L + el
          a = grp[el]
          for g in range(ngrp):
            msg_v[m, e, pl.ds(g * L, L)] = msg_v[m, e, pl.ds(g * L, L)] * a
        return 0
      lax.fori_loop(0, EB // L, _body, 0)

    def _fire_gather(m, j):
      pltpu.async_copy(xflat_hbm.at[idx_v.at[j]], msg_v.at[m], gsem[m])

    def _wait_gather(m, j):
      pltpu.make_async_copy(xflat_hbm.at[idx_v.at[j]], msg_v.at[m],
                            gsem[m]).wait()

    def _fire_scatter(m, j):
      pltpu.async_copy(msg_v.at[m], agg_sh.at[pkd_v.at[j, 1]], ssem[m],
                       add=True)

    def _wait_scatter(m, j):
      pltpu.make_async_copy(msg_v.at[m], agg_sh.at[pkd_v.at[j, 1]],
                            ssem[m]).wait()

    def _fire_load(j2, b):
      pltpu.async_copy(pkd_hbm.at[pl.ds(b, 2)], pkd_v.at[pl.ds(j2, 2)],
                       lsem[0])
      pltpu.async_copy(attr_hbm.at[pl.ds(b, 2)], attr_v.at[pl.ds(j2, 2)],
                       lsem[1])

    def _wait_load(j2, b):
      pltpu.make_async_copy(pkd_hbm.at[pl.ds(b, 2)], pkd_v.at[pl.ds(j2, 2)],
                            lsem[0]).wait()
      pltpu.make_async_copy(attr_hbm.at[pl.ds(b, 2)], attr_v.at[pl.ds(j2, 2)],
                            lsem[1]).wait()

    _fire_load(0, b0)
    _wait_load(0, b0)
    _compute_idx(0)
    _compute_idx(1)
    _fire_gather(0, 0)
    _fire_gather(1, 1)

    def _roll(i, _):
      base = b0 + 2 + i * 4
      _fire_load(2, base)
      _wait_gather(0, 0); _scale(0, 0); _fire_scatter(0, 0)
      _wait_gather(1, 1); _scale(1, 1); _fire_scatter(1, 1)
      _wait_load(2, base)
      _compute_idx(2); _compute_idx(3)
      _wait_scatter(0, 0); _fire_gather(0, 2)
      _wait_scatter(1, 1); _fire_gather(1, 3)
      _fire_load(0, base + 2)
      _wait_gather(0, 2); _scale(0, 2); _fire_scatter(0, 2)
      _wait_gather(1, 3); _scale(1, 3); _fire_scatter(1, 3)
      _wait_load(0, base + 2)
      _compute_idx(0); _compute_idx(1)
      _wait_scatter(0, 2); _fire_gather(0, 0)
      _wait_scatter(1, 3); _fire_gather(1, 1)
      return 0

    lax.fori_loop(0, niter, _roll, 0)

    _wait_gather(0, 0); _scale(0, 0); _fire_scatter(0, 0)
    _wait_gather(1, 1); _scale(1, 1); _fire_scatter(1, 1)
    _wait_scatter(0, 0)
    _wait_scatter(1, 1)

    @pl.when(s < blk_extra)
    def _leftover():
      b = NS * blk_base + s
      pltpu.sync_copy(pkd_hbm.at[pl.ds(b, 1)], pkd_v.at[pl.ds(0, 1)])
      pltpu.sync_copy(attr_hbm.at[pl.ds(b, 1)], attr_v.at[pl.ds(0, 1)])
      _compute_idx(0)
      pltpu.sync_copy(xflat_hbm.at[idx_v.at[0]], msg_v.at[0])
      _scale(0, 0)
      pltpu.sync_copy(msg_v.at[0], agg_sh.at[pkd_v.at[0, 1]], add=True)

    plsc.subcore_barrier()

    pltpu.sync_copy(agg_sh.at[pl.ds(r0, rpt)],
                    out_hbm.at[c, pl.ds(r0, rpt)])

    @pl.when(s == NS - 1)
    def _write_tail():
      pltpu.sync_copy(agg_sh.at[pl.ds(NS * rpt, tail)],
                      out_hbm.at[c, pl.ds(NS * rpt, tail)])

  return spmm(xflat, pkd, attrb)


def _dense_tc(agg2, w2, bias2d, n_nodes, d_out, block_m):
  d_half = agg2.shape[2]

  def body(a_ref, w_ref, b_ref, o_ref):
    y = jax.lax.dot_general(
        a_ref[0], w_ref[0], (((1,), (0,)), ((), ())),
        precision=lax.Precision.DEFAULT,
        preferred_element_type=jnp.float32)
    y = y + jax.lax.dot_general(
        a_ref[1], w_ref[1], (((1,), (0,)), ((), ())),
        precision=lax.Precision.DEFAULT,
        preferred_element_type=jnp.float32)
    y = y + b_ref[...]
    n2 = jnp.sum(y * y, axis=-1, keepdims=True)
    denom = jnp.maximum(jnp.sqrt(n2), 1e-12)
    o_ref[...] = y / denom

  grid = (n_nodes // block_m,)
  return pl.pallas_call(
      body,
      grid=grid,
      in_specs=[
          pl.BlockSpec((NC, block_m, d_half), lambda i: (0, i, 0)),
          pl.BlockSpec((NC, d_half, d_out), lambda i: (0, 0, 0)),
          pl.BlockSpec((1, d_out), lambda i: (0, 0)),
      ],
      out_specs=pl.BlockSpec((block_m, d_out), lambda i: (i, 0)),
      out_shape=jax.ShapeDtypeStruct((n_nodes, d_out), jnp.float32),
  )(agg2, w2, bias2d)


@jax.jit
def kernel(x, edge_index, edge_attr, W, b):
  n_nodes, d_in = x.shape
  n_edges = edge_index.shape[1]
  d_out = W.shape[1]
  d_half = d_in // NC

  xflat = x.reshape(n_nodes * NC, d_half)
  row = edge_index[0].astype(jnp.int32)
  col = edge_index[1].astype(jnp.int32)
  pkd = jnp.stack([col.reshape(-1, EB), row.reshape(-1, EB)], axis=1)
  attrb = edge_attr.astype(jnp.float32).reshape(-1, 1, EB)

  agg2 = _spmm_sc(xflat, pkd, attrb, n_nodes, n_edges, d_half)

  w2 = W.reshape(NC, d_half, d_out)
  bias2d = b.reshape(1, d_out)
  return _dense_tc(agg2, w2, bias2d, n_nodes, d_out, block_m=2000)
